# baseline (speedup 1.0000x reference)
import functools

import jax
import jax.numpy as jnp
from jax import lax
from jax.experimental import pallas as pl
from jax.experimental.pallas import tpu as pltpu
from jax.experimental.pallas import tpu_sc as plsc

_VOC = 100000
_E = 32
_F = 100
_H = 100
_B = 4096
_L = 200
_K = 20

_GATHER_W = 128
_BB = 128


def _sc_gather(table, idx_flat):
    n = idx_flat.shape[0]
    mesh = plsc.VectorSubcoreMesh(core_axis_name="c", subcore_axis_name="s")

    @functools.partial(
        pl.kernel,
        out_type=jax.ShapeDtypeStruct((n, _E), table.dtype),
        mesh=mesh,
        compiler_params=pltpu.CompilerParams(use_tc_tiling_on_sc=False),
    )
    def gather_kernel(t_hbm, i_hbm, o_hbm):
        def body(i_vmem, o_vmem):
            pltpu.sync_copy(t_hbm.at[i_vmem.at[0]], o_vmem)

        pltpu.emit_pipeline(
            body,
            grid=(n // _GATHER_W,),
            in_specs=[pl.BlockSpec((1, _GATHER_W), index_map=lambda i: (0, i))],
            out_specs=[
                pl.BlockSpec((_GATHER_W, _E), index_map=lambda i: (i, 0)),
            ],
            core_axis_name=("c", "s"),
            dimension_semantics=(pltpu.PARALLEL,),
        )(i_hbm, o_hbm)

    return gather_kernel(table, idx_flat.reshape(1, n))


def _bdot(a, b):
    return jnp.dot(a.astype(jnp.bfloat16), b.astype(jnp.bfloat16),
                   preferred_element_type=jnp.float32)


def _conv3tap(xf, w_tap, bias, bb, cout):
    cin = xf.shape[-1]
    pad = (-cin) % 8
    x3 = xf.astype(jnp.bfloat16).reshape(bb, _L, cin)
    z = jnp.zeros((bb, 1, cin), jnp.bfloat16)
    xm = jnp.concatenate([z, x3[:, : _L - 1, :]], axis=1)
    xp = jnp.concatenate([x3[:, 1:, :], z], axis=1)
    w = w_tap.astype(jnp.bfloat16)
    if pad:
        zx = jnp.zeros((bb, _L, pad), jnp.bfloat16)
        zw = jnp.zeros((pad, cout), jnp.bfloat16)
        xcat = jnp.concatenate([xm, zx, x3, zx, xp, zx], axis=2)
        wcat = jnp.concatenate([w[0], zw, w[1], zw, w[2], zw], axis=0)
    else:
        xcat = jnp.concatenate([xm, x3, xp], axis=2)
        wcat = jnp.concatenate([w[0], w[1], w[2]], axis=0)
    xcat = xcat.reshape(bb * _L, 3 * (cin + pad))
    y = jnp.dot(xcat, wcat, preferred_element_type=jnp.float32)
    return y.reshape(bb, _L, cout) + bias


def _tc_body(embed_ref, w1_ref, b1_ref, gw_ref, gb_ref, w2_ref,
             b2_ref, wl_ref, bl_ref, w3l_ref, w3g_ref, b3_ref, w4_ref, b4_ref,
             t_ref):
    bb = t_ref.shape[0]
    xf = embed_ref[...]

    c1 = jax.nn.relu(_conv3tap(xf, w1_ref[...], b1_ref[...], bb, _F))
    gm = jnp.zeros((bb, _F), jnp.float32)
    for l in range(_L):
        gm = gm + c1[:, l, :]
    g = gm * jnp.float32(1.0 / _L)
    g = jax.nn.relu(_bdot(g, gw_ref[...]) + gb_ref[...])

    c1f = c1.reshape(bb * _L, _F)
    loc = jax.nn.relu(_conv3tap(c1f, w2_ref[...], b2_ref[...], bb, _H))
    locf = loc.reshape(bb * _L, _H)
    loc2 = jax.nn.relu(_conv3tap(locf, wl_ref[...], bl_ref[...], bb, _H))

    gb_full = jnp.broadcast_to(g.reshape(bb, 1, _H), (bb, _L, _H))
    comb = jnp.concatenate([gb_full, loc2], axis=2).reshape(bb * _L, 2 * _H)
    w3cat = jnp.concatenate([w3g_ref[...], w3l_ref[...]], axis=0)
    h = jax.nn.relu(_bdot(comb, w3cat).reshape(bb, _L, _F) + b3_ref[...])
    logits = (_bdot(h.reshape(bb * _L, _F), w4_ref[...]).reshape(bb, _L)
              + b4_ref[...])

    neg = jnp.float32(-3.0e38)
    pos = lax.broadcasted_iota(jnp.int32, (bb, _L), 1)
    work = logits
    thr = None
    for _ in range(_K):
        thr = jnp.max(work, axis=1, keepdims=True)
        hit = work >= thr
        first = jnp.min(jnp.where(hit, pos, _L), axis=1, keepdims=True)
        work = jnp.where(pos == first, neg, work)
    t = (logits >= thr).astype(jnp.float32)
    t_ref[...] = t


_BB2 = 256


def _head_body(emb2_ref, t_ref, fc1_ref, fc1b_ref, hw_ref, hb_ref, out_ref):
    bb = out_ref.shape[0]
    emb2 = emb2_ref[...].astype(jnp.float32).reshape(bb, _L, _E)
    t = t_ref[...]
    s = jnp.sum(emb2 * t[:, :, None], axis=1) * jnp.float32(1.0 / _L)
    mlp = jax.nn.relu(jnp.dot(s, fc1_ref[...],
                              preferred_element_type=jnp.float32,
                              precision=lax.Precision.HIGHEST)
                      + fc1b_ref[...])
    o = jnp.sum(mlp * hw_ref[...], axis=1, keepdims=True) + hb_ref[...]
    out_ref[...] = jax.nn.sigmoid(o)


def _full_spec(shape):
    nd = len(shape)
    return pl.BlockSpec(shape, lambda i, _nd=nd: (0,) * _nd)


def _tc_forward(embed_flat, weights, interpret=False):
    grid = _B // _BB
    in_specs = [
        pl.BlockSpec((_BB * _L, _E), lambda i: (i, 0)),
    ] + [_full_spec(w.shape) for w in weights]
    out_specs = pl.BlockSpec((_BB, _L), lambda i: (i, 0))
    out_shape = jax.ShapeDtypeStruct((_B, _L), jnp.float32)
    return pl.pallas_call(
        _tc_body,
        grid=(grid,),
        in_specs=in_specs,
        out_specs=out_specs,
        out_shape=out_shape,
        interpret=interpret,
    )(embed_flat, *weights)


def _head_forward(emb2_flat, t, weights, interpret=False):
    grid = _B // _BB2
    in_specs = [
        pl.BlockSpec((_BB2 * _L, _E), lambda i: (i, 0)),
        pl.BlockSpec((_BB2, _L), lambda i: (i, 0)),
    ] + [_full_spec(w.shape) for w in weights]
    out_specs = pl.BlockSpec((_BB2, 1), lambda i: (i, 0))
    out_shape = jax.ShapeDtypeStruct((_B, 1), jnp.float32)
    return pl.pallas_call(
        _head_body,
        grid=(grid,),
        in_specs=in_specs,
        out_specs=out_specs,
        out_shape=out_shape,
        interpret=interpret,
    )(emb2_flat, t, *weights)


def _prep_weights(conv1_w, conv1_b, glob_w, glob_b, conv2_w, conv2_b, loc_w,
                  loc_b, conv3_w, conv3_b, conv4_w, conv4_b, fc1_w, fc1_b,
                  head_w, head_b):
    w1 = jnp.transpose(conv1_w, (2, 1, 0))
    w2 = jnp.transpose(conv2_w, (2, 1, 0))
    wl = jnp.transpose(loc_w, (2, 1, 0))
    w3g = jnp.transpose(conv3_w[:, :_H, 0])
    w3l = jnp.transpose(conv3_w[:, _H:, 0])
    return (
        w1, conv1_b.reshape(1, 1, _F),
        jnp.transpose(glob_w), glob_b.reshape(1, _H),
        w2, conv2_b.reshape(1, 1, _H),
        wl, loc_b.reshape(1, 1, _H),
        w3l, w3g, conv3_b.reshape(1, 1, _F),
        conv4_w[0], conv4_b.reshape(1, 1),
        jnp.transpose(fc1_w), fc1_b.reshape(1, _H),
        head_w.reshape(1, _H), head_b.reshape(1, 1),
    )


def kernel(x, emb_ti, conv1_w, conv1_b, glob_w, glob_b, conv2_w, conv2_b,
           loc_w, loc_b, conv3_w, conv3_b, conv4_w, conv4_b, emb_d, fc1_w,
           fc1_b, head_w, head_b):
    idx = x.reshape(-1).astype(jnp.int32)
    embed_flat = _sc_gather(emb_ti.astype(jnp.bfloat16), idx)
    emb2_flat = _sc_gather(emb_d.astype(jnp.bfloat16), idx)
    weights = _prep_weights(conv1_w, conv1_b, glob_w, glob_b, conv2_w,
                            conv2_b, loc_w, loc_b, conv3_w, conv3_b, conv4_w,
                            conv4_b, fc1_w, fc1_b, head_w, head_b)
    t = _tc_forward(embed_flat, weights[:13])
    out = _head_forward(emb2_flat, t, weights[13:])
    return out, t


# speedup vs baseline: 1.0904x; 1.0904x over previous
import functools

import jax
import jax.numpy as jnp
from jax import lax
from jax.experimental import pallas as pl
from jax.experimental.pallas import tpu as pltpu
from jax.experimental.pallas import tpu_sc as plsc

_VOC = 100000
_E = 32
_F = 100
_H = 100
_B = 4096
_L = 200
_K = 20

_GATHER_W = 128
_BB = 64


def _sc_gather(table, idx_flat):
    n = idx_flat.shape[0]
    mesh = plsc.VectorSubcoreMesh(core_axis_name="c", subcore_axis_name="s")

    @functools.partial(
        pl.kernel,
        out_type=jax.ShapeDtypeStruct((n, _E), table.dtype),
        mesh=mesh,
        compiler_params=pltpu.CompilerParams(use_tc_tiling_on_sc=False),
    )
    def gather_kernel(t_hbm, i_hbm, o_hbm):
        def body(i_vmem, o_vmem):
            pltpu.sync_copy(t_hbm.at[i_vmem.at[0]], o_vmem)

        pltpu.emit_pipeline(
            body,
            grid=(n // _GATHER_W,),
            in_specs=[pl.BlockSpec((1, _GATHER_W), index_map=lambda i: (0, i))],
            out_specs=[
                pl.BlockSpec((_GATHER_W, _E), index_map=lambda i: (i, 0)),
            ],
            core_axis_name=("c", "s"),
            dimension_semantics=(pltpu.PARALLEL,),
        )(i_hbm, o_hbm)

    return gather_kernel(table, idx_flat.reshape(1, n))


def _bdot(a, b):
    return jnp.dot(a.astype(jnp.bfloat16), b.astype(jnp.bfloat16),
                   preferred_element_type=jnp.float32)


def _conv3tap(xf, w_tap, bias, bb, cout):
    cin = xf.shape[-1]
    pad = (-cin) % 8
    x3 = xf.astype(jnp.bfloat16).reshape(bb, _L, cin)
    z = jnp.zeros((bb, 1, cin), jnp.bfloat16)
    xm = jnp.concatenate([z, x3[:, : _L - 1, :]], axis=1)
    xp = jnp.concatenate([x3[:, 1:, :], z], axis=1)
    w = w_tap.astype(jnp.bfloat16)
    if pad:
        zx = jnp.zeros((bb, _L, pad), jnp.bfloat16)
        zw = jnp.zeros((pad, cout), jnp.bfloat16)
        xcat = jnp.concatenate([xm, zx, x3, zx, xp, zx], axis=2)
        wcat = jnp.concatenate([w[0], zw, w[1], zw, w[2], zw], axis=0)
    else:
        xcat = jnp.concatenate([xm, x3, xp], axis=2)
        wcat = jnp.concatenate([w[0], w[1], w[2]], axis=0)
    xcat = xcat.reshape(bb * _L, 3 * (cin + pad))
    y = jnp.dot(xcat, wcat, preferred_element_type=jnp.float32)
    return y.reshape(bb, _L, cout) + bias


def _tc_body(embed_ref, w1_ref, b1_ref, gw_ref, gb_ref, w2_ref,
             b2_ref, wl_ref, bl_ref, w3l_ref, w3g_ref, b3_ref, w4_ref, b4_ref,
             t_ref):
    bb = t_ref.shape[0]
    xf = embed_ref[...]

    c1 = jax.nn.relu(_conv3tap(xf, w1_ref[...], b1_ref[...], bb, _F))
    gm = jnp.zeros((bb, _F), jnp.float32)
    for l in range(_L):
        gm = gm + c1[:, l, :]
    g = gm * jnp.float32(1.0 / _L)
    g = jax.nn.relu(_bdot(g, gw_ref[...]) + gb_ref[...])

    c1f = c1.reshape(bb * _L, _F)
    loc = jax.nn.relu(_conv3tap(c1f, w2_ref[...], b2_ref[...], bb, _H))
    locf = loc.reshape(bb * _L, _H)
    loc2 = jax.nn.relu(_conv3tap(locf, wl_ref[...], bl_ref[...], bb, _H))

    gb_full = jnp.broadcast_to(g.reshape(bb, 1, _H), (bb, _L, _H))
    comb = jnp.concatenate([gb_full, loc2], axis=2).reshape(bb * _L, 2 * _H)
    w3cat = jnp.concatenate([w3g_ref[...], w3l_ref[...]], axis=0)
    h = jax.nn.relu(_bdot(comb, w3cat).reshape(bb, _L, _F) + b3_ref[...])
    logits = (_bdot(h.reshape(bb * _L, _F), w4_ref[...]).reshape(bb, _L)
              + b4_ref[...])

    neg = jnp.float32(-3.0e38)
    pos = lax.broadcasted_iota(jnp.int32, (bb, _L), 1)
    work = logits
    thr = None
    for _ in range(_K):
        thr = jnp.max(work, axis=1, keepdims=True)
        hit = work >= thr
        first = jnp.min(jnp.where(hit, pos, _L), axis=1, keepdims=True)
        work = jnp.where(pos == first, neg, work)
    t = (logits >= thr).astype(jnp.float32)
    t_ref[...] = t


_BB2 = 256


def _head_body(emb2_ref, t_ref, fc1_ref, fc1b_ref, hw_ref, hb_ref, out_ref):
    bb = out_ref.shape[0]
    emb2 = emb2_ref[...].astype(jnp.float32).reshape(bb, _L, _E)
    t = t_ref[...]
    s = jnp.sum(emb2 * t[:, :, None], axis=1) * jnp.float32(1.0 / _L)
    mlp = jax.nn.relu(jnp.dot(s, fc1_ref[...],
                              preferred_element_type=jnp.float32,
                              precision=lax.Precision.HIGHEST)
                      + fc1b_ref[...])
    o = jnp.sum(mlp * hw_ref[...], axis=1, keepdims=True) + hb_ref[...]
    out_ref[...] = jax.nn.sigmoid(o)


def _full_spec(shape):
    nd = len(shape)
    return pl.BlockSpec(shape, lambda i, _nd=nd: (0,) * _nd)


def _tc_forward(embed_flat, weights, interpret=False):
    grid = _B // _BB
    in_specs = [
        pl.BlockSpec((_BB * _L, _E), lambda i: (i, 0)),
    ] + [_full_spec(w.shape) for w in weights]
    out_specs = pl.BlockSpec((_BB, _L), lambda i: (i, 0))
    out_shape = jax.ShapeDtypeStruct((_B, _L), jnp.float32)
    return pl.pallas_call(
        _tc_body,
        grid=(grid,),
        in_specs=in_specs,
        out_specs=out_specs,
        out_shape=out_shape,
        interpret=interpret,
    )(embed_flat, *weights)


def _head_forward(emb2_flat, t, weights, interpret=False):
    grid = _B // _BB2
    in_specs = [
        pl.BlockSpec((_BB2 * _L, _E), lambda i: (i, 0)),
        pl.BlockSpec((_BB2, _L), lambda i: (i, 0)),
    ] + [_full_spec(w.shape) for w in weights]
    out_specs = pl.BlockSpec((_BB2, 1), lambda i: (i, 0))
    out_shape = jax.ShapeDtypeStruct((_B, 1), jnp.float32)
    return pl.pallas_call(
        _head_body,
        grid=(grid,),
        in_specs=in_specs,
        out_specs=out_specs,
        out_shape=out_shape,
        interpret=interpret,
    )(emb2_flat, t, *weights)


def _prep_weights(conv1_w, conv1_b, glob_w, glob_b, conv2_w, conv2_b, loc_w,
                  loc_b, conv3_w, conv3_b, conv4_w, conv4_b, fc1_w, fc1_b,
                  head_w, head_b):
    w1 = jnp.transpose(conv1_w, (2, 1, 0))
    w2 = jnp.transpose(conv2_w, (2, 1, 0))
    wl = jnp.transpose(loc_w, (2, 1, 0))
    w3g = jnp.transpose(conv3_w[:, :_H, 0])
    w3l = jnp.transpose(conv3_w[:, _H:, 0])
    return (
        w1, conv1_b.reshape(1, 1, _F),
        jnp.transpose(glob_w), glob_b.reshape(1, _H),
        w2, conv2_b.reshape(1, 1, _H),
        wl, loc_b.reshape(1, 1, _H),
        w3l, w3g, conv3_b.reshape(1, 1, _F),
        conv4_w[0], conv4_b.reshape(1, 1),
        jnp.transpose(fc1_w), fc1_b.reshape(1, _H),
        head_w.reshape(1, _H), head_b.reshape(1, 1),
    )


def kernel(x, emb_ti, conv1_w, conv1_b, glob_w, glob_b, conv2_w, conv2_b,
           loc_w, loc_b, conv3_w, conv3_b, conv4_w, conv4_b, emb_d, fc1_w,
           fc1_b, head_w, head_b):
    idx = x.reshape(-1).astype(jnp.int32)
    embed_flat = _sc_gather(emb_ti.astype(jnp.bfloat16), idx)
    emb2_flat = _sc_gather(emb_d.astype(jnp.bfloat16), idx)
    weights = _prep_weights(conv1_w, conv1_b, glob_w, glob_b, conv2_w,
                            conv2_b, loc_w, loc_b, conv3_w, conv3_b, conv4_w,
                            conv4_b, fc1_w, fc1_b, head_w, head_b)
    t = _tc_forward(embed_flat, weights[:13])
    out = _head_forward(emb2_flat, t, weights[13:])
    return out, t
